# baseline (speedup 1.0000x reference)
import functools

import jax
import jax.numpy as jnp
from jax import lax
from jax.experimental import pallas as pl
from jax.experimental.pallas import tpu as pltpu
from jax.experimental.pallas import tpu_sc as plsc


def _sc_add_rel_pos(x2, t2, *, s, s2, d, nrows, maxrel, nw, cj):
    rows_per_w = s // nw
    nchunks = s2 // cj
    mesh = plsc.VectorSubcoreMesh(core_axis_name="c", subcore_axis_name="s")

    @functools.partial(
        pl.kernel,
        mesh=mesh,
        out_type=jax.ShapeDtypeStruct((s, s2 * d), jnp.float32),
        scratch_types=[
            pltpu.VMEM((nrows * d,), jnp.float32),
            pltpu.VMEM((cj * d,), jnp.float32),
        ],
    )
    def k(x_hbm, t_hbm, o_hbm, t_v, buf):
        wid = lax.axis_index("s") * 2 + lax.axis_index("c")
        pltpu.sync_copy(t_hbm, t_v)
        i0 = wid * rows_per_w

        def row_body(il, carry):
            i = i0 + il

            def chunk_body(c, carry2):
                j0 = c * cj
                pltpu.sync_copy(x_hbm.at[i, pl.ds(j0 * d, cj * d)], buf)

                def j_body(j, carry3):
                    rel = i - (j0 + j)
                    r = jnp.clip(rel, -maxrel, maxrel) + maxrel
                    base = r * d
                    for dd in range(0, d, 16):
                        buf[pl.ds(j * d + dd, 16)] = (
                            buf[pl.ds(j * d + dd, 16)]
                            + t_v[pl.ds(base + dd, 16)]
                        )
                    return carry3

                lax.fori_loop(0, cj, j_body, 0)
                pltpu.sync_copy(buf, o_hbm.at[i, pl.ds(j0 * d, cj * d)])
                return carry2

            lax.fori_loop(0, nchunks, chunk_body, 0)
            return carry

        lax.fori_loop(0, rows_per_w, row_body, 0)

    return k(x2, t2)


@jax.jit
def kernel(x, table):
    s, s2, d = x.shape
    nrows = table.shape[0]
    maxrel = (nrows - 1) // 2
    x2 = x.reshape(s, s2 * d)
    t2 = table.reshape(nrows * d)
    out = _sc_add_rel_pos(
        x2, t2, s=s, s2=s2, d=d, nrows=nrows, maxrel=maxrel, nw=32, cj=64
    )
    return out.reshape(s, s2, d)


# speedup vs baseline: 2.0972x; 2.0972x over previous
import functools

import jax
import jax.numpy as jnp
from jax import lax
from jax.experimental import pallas as pl
from jax.experimental.pallas import tpu as pltpu
from jax.experimental.pallas import tpu_sc as plsc

_NBUF = 4
_CJ = 32


def _sc_add_rel_pos(x2, t2, *, s, s2, d, nrows, maxrel, nw, cj):
    rows_per_w = s // nw
    nchunks = s2 // cj
    nchunk_tot = rows_per_w * nchunks
    mesh = plsc.VectorSubcoreMesh(core_axis_name="c", subcore_axis_name="s")

    @functools.partial(
        pl.kernel,
        mesh=mesh,
        out_type=jax.ShapeDtypeStruct((s, s2 * d), jnp.float32),
        scratch_types=(
            [pltpu.VMEM((nrows * d,), jnp.float32)]
            + [pltpu.VMEM((cj * d,), jnp.float32) for _ in range(_NBUF)]
            + [pltpu.SemaphoreType.DMA for _ in range(2 * _NBUF)]
        ),
    )
    def k(x_hbm, t_hbm, o_hbm, t_v, *bufs_and_sems):
        bufs = bufs_and_sems[:_NBUF]
        lsems = bufs_and_sems[_NBUF:2 * _NBUF]
        ssems = bufs_and_sems[2 * _NBUF:3 * _NBUF]
        wid = lax.axis_index("s") * 2 + lax.axis_index("c")
        pltpu.sync_copy(t_hbm, t_v)
        i0 = wid * rows_per_w

        def chunk_slice(cc):
            i = i0 + cc // nchunks
            j0 = (cc % nchunks) * cj
            return i, j0

        def start_load(cc, slot):
            i, j0 = chunk_slice(cc)
            pltpu.async_copy(
                x_hbm.at[i, pl.ds(j0 * d, cj * d)], bufs[slot], lsems[slot]
            )

        def wait_load(slot):
            pltpu.make_async_copy(
                x_hbm.at[0, pl.ds(0, cj * d)], bufs[slot], lsems[slot]
            ).wait()

        def start_store(cc, slot):
            i, j0 = chunk_slice(cc)
            pltpu.async_copy(
                bufs[slot], o_hbm.at[i, pl.ds(j0 * d, cj * d)], ssems[slot]
            )

        def wait_store(slot):
            pltpu.make_async_copy(
                bufs[slot], o_hbm.at[0, pl.ds(0, cj * d)], ssems[slot]
            ).wait()

        def compute(cc, slot):
            buf = bufs[slot]
            i, j0 = chunk_slice(cc)
            na = jnp.clip(i - maxrel - j0 + 1, 0, cj)
            nb = jnp.clip(i + maxrel - j0, 0, cj)
            for db in range(d // 256):
                off = db * 256
                thi = [
                    t_v[pl.ds(2 * maxrel * d + off + q * 16, 16)]
                    for q in range(16)
                ]

                @plsc.parallel_loop(0, na)
                def _(j):
                    for q in range(16):
                        a = j * d + off + q * 16
                        buf[pl.ds(a, 16)] = buf[pl.ds(a, 16)] + thi[q]

                tlo = [t_v[pl.ds(off + q * 16, 16)] for q in range(16)]

                @plsc.parallel_loop(nb, cj)
                def _(j):
                    for q in range(16):
                        a = j * d + off + q * 16
                        buf[pl.ds(a, 16)] = buf[pl.ds(a, 16)] + tlo[q]

            @plsc.parallel_loop(na, nb)
            def _(j):
                r = i - (j0 + j) + maxrel
                base = r * d
                for dd in range(0, d, 16):
                    buf[pl.ds(j * d + dd, 16)] = (
                        buf[pl.ds(j * d + dd, 16)] + t_v[pl.ds(base + dd, 16)]
                    )

        start_load(0, 0)
        start_load(1, 1)

        @pl.loop(0, nchunk_tot, step=_NBUF)
        def _(cc0):
            for b in range(_NBUF):
                cc = cc0 + b
                pslot = (b + 2) % _NBUF

                @pl.when(cc + 2 < nchunk_tot)
                def _():
                    @pl.when(cc >= 2)
                    def _():
                        wait_store(pslot)

                    start_load(cc + 2, pslot)

                wait_load(b)
                compute(cc, b)
                start_store(cc, b)

        for b in range(_NBUF):
            wait_store(b)

    return k(x2, t2)


@jax.jit
def kernel(x, table):
    s, s2, d = x.shape
    nrows = table.shape[0]
    maxrel = (nrows - 1) // 2
    x2 = x.reshape(s, s2 * d)
    t2 = table.reshape(nrows * d)
    out = _sc_add_rel_pos(
        x2, t2, s=s, s2=s2, d=d, nrows=nrows, maxrel=maxrel, nw=32, cj=_CJ
    )
    return out.reshape(s, s2, d)


# speedup vs baseline: 6.0747x; 2.8966x over previous
import functools

import jax
import jax.numpy as jnp
from jax import lax
from jax.experimental import pallas as pl
from jax.experimental.pallas import tpu as pltpu
from jax.experimental.pallas import tpu_sc as plsc

_NBUF = 4
_CJ = 32


def _sc_add_rel_pos(x, table, *, s, s2, d, nrows, maxrel, nw, cj):
    rows_per_w = s // nw
    nchunks = s2 // cj
    nchunk_tot = rows_per_w * nchunks
    mesh = plsc.VectorSubcoreMesh(core_axis_name="c", subcore_axis_name="s")

    @functools.partial(
        pl.kernel,
        mesh=mesh,
        out_type=jax.ShapeDtypeStruct((s, s2, d), jnp.float32),
        scratch_types=(
            [pltpu.VMEM((nrows, d), jnp.float32)]
            + [pltpu.VMEM((cj, d), jnp.float32) for _ in range(_NBUF)]
            + [pltpu.SemaphoreType.DMA for _ in range(2 * _NBUF)]
        ),
    )
    def k(x_hbm, t_hbm, o_hbm, t_v, *bufs_and_sems):
        bufs = bufs_and_sems[:_NBUF]
        lsems = bufs_and_sems[_NBUF:2 * _NBUF]
        ssems = bufs_and_sems[2 * _NBUF:3 * _NBUF]
        wid = lax.axis_index("s") * 2 + lax.axis_index("c")
        pltpu.sync_copy(t_hbm, t_v)
        i0 = wid * rows_per_w

        def chunk_slice(cc):
            i = i0 + cc // nchunks
            j0 = (cc % nchunks) * cj
            return i, j0

        def start_load(cc, slot):
            i, j0 = chunk_slice(cc)
            pltpu.async_copy(
                x_hbm.at[i, pl.ds(j0, cj)], bufs[slot], lsems[slot]
            )

        def wait_load(slot):
            pltpu.make_async_copy(
                x_hbm.at[0, pl.ds(0, cj)], bufs[slot], lsems[slot]
            ).wait()

        def start_store(cc, slot):
            i, j0 = chunk_slice(cc)
            pltpu.async_copy(
                bufs[slot], o_hbm.at[i, pl.ds(j0, cj)], ssems[slot]
            )

        def wait_store(slot):
            pltpu.make_async_copy(
                bufs[slot], o_hbm.at[0, pl.ds(0, cj)], ssems[slot]
            ).wait()

        def compute(cc, slot):
            buf = bufs[slot]
            i, j0 = chunk_slice(cc)
            na = jnp.clip(i - maxrel - j0 + 1, 0, cj)
            nb = jnp.clip(i + maxrel - j0, 0, cj)
            for db in range(d // 256):
                off = db * 256
                thi = [
                    t_v[2 * maxrel, pl.ds(off + q * 16, 16)]
                    for q in range(16)
                ]

                @plsc.parallel_loop(0, na)
                def _(j):
                    for q in range(16):
                        sl = pl.ds(off + q * 16, 16)
                        buf[j, sl] = buf[j, sl] + thi[q]

                tlo = [t_v[0, pl.ds(off + q * 16, 16)] for q in range(16)]

                @plsc.parallel_loop(nb, cj)
                def _(j):
                    for q in range(16):
                        sl = pl.ds(off + q * 16, 16)
                        buf[j, sl] = buf[j, sl] + tlo[q]

            @plsc.parallel_loop(na, nb)
            def _(j):
                r = i - (j0 + j) + maxrel
                for dd in range(0, d, 16):
                    sl = pl.ds(dd, 16)
                    buf[j, sl] = buf[j, sl] + t_v[r, sl]

        start_load(0, 0)
        start_load(1, 1)

        @pl.loop(0, nchunk_tot, step=_NBUF)
        def _(cc0):
            for b in range(_NBUF):
                cc = cc0 + b
                pslot = (b + 2) % _NBUF

                @pl.when(cc + 2 < nchunk_tot)
                def _():
                    @pl.when(cc >= 2)
                    def _():
                        wait_store(pslot)

                    start_load(cc + 2, pslot)

                wait_load(b)
                compute(cc, b)
                start_store(cc, b)

        for b in range(_NBUF):
            wait_store(b)

    return k(x, table)


@jax.jit
def kernel(x, table):
    s, s2, d = x.shape
    nrows = table.shape[0]
    maxrel = (nrows - 1) // 2
    return _sc_add_rel_pos(
        x, table, s=s, s2=s2, d=d, nrows=nrows, maxrel=maxrel, nw=32, cj=_CJ
    )
